# baseline (speedup 1.0000x reference)
import jax
import jax.numpy as jnp
from jax import lax
from jax.experimental import pallas as pl
from jax.experimental.pallas import tpu as pltpu
from jax.experimental.pallas import tpu_sc as plsc

D = 1024
NC = 2
NS = 16
NW = NC * NS
B = 4 * 8192
BPW = B // NW
C = 16
NCH = BPW // C
NBUF = 3
NHOP = 2


def _pe_body(idx_hbm, table_hbm, out_hbm, idx_v, rows_v, rows_s,
             gsem, dsem, csem, hsem):
    sid = lax.axis_index("s")
    wid = sid * NC + lax.axis_index("c")
    base = wid * BPW
    pltpu.sync_copy(idx_hbm.at[wid], idx_v)
    my_s = rows_s.at[sid]

    def out_ref(j):
        return out_hbm.at[pl.ds(base + j * C, C)]

    def wait_once(rec):
        if rec is not None and not rec["waited"]:
            rec["h"].wait()
            rec["waited"] = True

    gather = [None] * NBUF
    consumer = [None] * NBUF
    hbmout = [None] * NHOP
    crossq = []

    def start_gather(j):
        slot = j % NBUF
        wait_once(consumer[slot])
        gather[slot] = pltpu.async_copy(
            table_hbm.at[idx_v.at[j]], rows_v.at[slot], gsem)

    def launch_hbm_leg():
        phs, pcb, pj = crossq.pop(0)
        wait_once(pcb)
        hbmout[phs] = {
            "h": pltpu.async_copy(my_s.at[phs], out_ref(pj), hsem),
            "waited": False}

    for j in range(min(NBUF, NCH)):
        start_gather(j)

    for j in range(NCH):
        slot = j % NBUF
        gather[slot].wait()
        if j % 2 == 0:
            consumer[slot] = {
                "h": pltpu.async_copy(rows_v.at[slot], out_ref(j), dsem),
                "waited": False}
        else:
            hs = (j // 2) % NHOP
            wait_once(hbmout[hs])
            cb = {"h": pltpu.async_copy(rows_v.at[slot], my_s.at[hs], csem),
                  "waited": False}
            consumer[slot] = cb
            crossq.append((hs, cb, j))
            if len(crossq) > 1:
                launch_hbm_leg()
        nj = j + NBUF
        if nj < NCH:
            start_gather(nj)
    while crossq:
        launch_hbm_leg()
    for slot in range(NBUF):
        wait_once(consumer[slot])
    for hs in range(NHOP):
        wait_once(hbmout[hs])


def kernel(x, table):
    idx = x.reshape(NW, NCH, C).astype(jnp.int32)
    mesh = plsc.VectorSubcoreMesh(core_axis_name="c", subcore_axis_name="s")
    out = pl.kernel(
        _pe_body,
        mesh=mesh,
        out_type=jax.ShapeDtypeStruct((B, D), jnp.float32),
        scratch_types=[
            pltpu.VMEM((NCH, C), jnp.int32),
            pltpu.VMEM((NBUF, C, D), jnp.float32),
            pltpu.VMEM_SHARED((NS, NHOP, C, D), jnp.float32),
            pltpu.SemaphoreType.DMA,
            pltpu.SemaphoreType.DMA,
            pltpu.SemaphoreType.DMA,
            pltpu.SemaphoreType.DMA,
        ],
    )(idx, table)
    return out.reshape(x.shape[0], x.shape[1], D)
